# Initial kernel scaffold; baseline (speedup 1.0000x reference)
import jax
import jax.numpy as jnp
from jax.experimental import pallas as pl


def kernel(x, edge_index, edge_attr, Ws, bs, W, b):
    raise NotImplementedError("write your pallas kernel here")


# speedup vs baseline: 378.9175x; 378.9175x over previous
import functools

import jax
import jax.numpy as jnp
from jax import lax
from jax.experimental import pallas as pl
from jax.experimental.pallas import tpu as pltpu
from jax.experimental.pallas import tpu_sc as plsc


def _scores(edge_attr, wpad):
    M, D = edge_attr.shape
    RB = 2560
    grid = M // RB

    def body(ea_ref, w_ref, o_ref):
        o_ref[...] = jnp.dot(ea_ref[...], w_ref[...],
                             preferred_element_type=jnp.float32)

    return pl.pallas_call(
        body,
        grid=(grid,),
        in_specs=[
            pl.BlockSpec((RB, D), lambda i: (i, 0)),
            pl.BlockSpec((D, 8), lambda i: (0, 0)),
        ],
        out_specs=pl.BlockSpec((RB, 8), lambda i: (i, 0)),
        out_shape=jax.ShapeDtypeStruct((M, 8), jnp.float32),
    )(edge_attr, wpad)


def _softmax(s2d, bs):
    def body(s_ref, bs_ref, o_ref):
        s = s_ref[...] + bs_ref[0, 0]
        m = jnp.max(s)
        e = jnp.exp(s - m)
        o_ref[...] = e / jnp.sum(e)

    return pl.pallas_call(
        body,
        in_specs=[
            pl.BlockSpec(s2d.shape, lambda: (0, 0)),
            pl.BlockSpec(memory_space=pltpu.SMEM),
        ],
        out_specs=pl.BlockSpec(s2d.shape, lambda: (0, 0)),
        out_shape=jax.ShapeDtypeStruct(s2d.shape, jnp.float32),
    )(s2d, bs.reshape(1, 1))


def _kruskal(u_s, v_s, n_nodes, n_pad):
    M = u_s.shape[0]
    CH = 2000
    assert M % CH == 0
    n_stop = n_nodes - 1
    dummy = n_pad - 1

    mesh = plsc.VectorSubcoreMesh(core_axis_name="c", subcore_axis_name="s")

    def s0(vec):
        return vec[0]

    def rd(ref, i):
        return s0(plsc.load_gather(ref, [jnp.full((16,), i, jnp.int32)]))

    def body(u_hbm, v_hbm, cu_hbm, cv_hbm, deg_hbm,
             parent, rank, cu_v, cv_v, deg_v, uc, vc, sem):
        cid = lax.axis_index("c")
        sid = lax.axis_index("s")
        lane0 = lax.iota(jnp.int32, 16) == 0

        def wr(ref, i, val):
            plsc.store_scatter(ref, [jnp.full((16,), i, jnp.int32)],
                               jnp.full((16,), val, ref.dtype), mask=lane0)

        @pl.when((cid == 0) & (sid == 0))
        def _():
            iota16 = lax.iota(jnp.int32, 16)
            zero16f = jnp.zeros((16,), jnp.float32)
            zero16i = jnp.zeros((16,), jnp.int32)
            dummy16 = jnp.full((16,), dummy, jnp.int32)

            def init_body(i, carry):
                base = i * 16
                parent[pl.ds(base, 16)] = iota16 + base
                rank[pl.ds(base, 16)] = zero16i
                cu_v[pl.ds(base, 16)] = dummy16
                cv_v[pl.ds(base, 16)] = dummy16
                deg_v[pl.ds(base, 16)] = zero16f
                return carry

            lax.fori_loop(0, n_pad // 16, init_body, 0)

            def find(n):
                p = rd(parent, n)

                def cond(c):
                    node, par = c
                    return par != node

                def step(c):
                    node, par = c
                    gp = rd(parent, par)
                    wr(parent, node, gp)
                    return par, gp

                node, _ = lax.while_loop(cond, step, (n, p))
                return node

            def inner(c):
                j, cnt = c
                u = rd(uc, j)
                v = rd(vc, j)
                ru = find(u)
                rv = find(v)
                acc = ru != rv

                @pl.when(acc)
                def _():
                    ra = rd(rank, ru)
                    rb = rd(rank, rv)
                    child = jnp.where(ra < rb, ru, rv)
                    root = jnp.where(ra < rb, rv, ru)
                    wr(parent, child, root)

                    @pl.when(ra == rb)
                    def _():
                        wr(rank, ru, ra + 1)

                    wr(cu_v, cnt, u)
                    wr(cv_v, cnt, v)
                    wr(deg_v, v, rd(deg_v, v) + 1.0)

                return j + 1, cnt + acc.astype(jnp.int32)

            def inner_cond(c):
                j, cnt = c
                return (j < CH) & (cnt < n_stop)

            def outer(c):
                pos, cnt = c
                pos = pl.multiple_of(pos, CH)
                pltpu.async_copy(u_hbm.at[pl.ds(pos, CH)], uc, sem).wait()
                pltpu.async_copy(v_hbm.at[pl.ds(pos, CH)], vc, sem).wait()
                _, cnt = lax.while_loop(inner_cond, inner, (0, cnt))
                return pos + CH, cnt

            def outer_cond(c):
                pos, cnt = c
                return (pos < M) & (cnt < n_stop)

            lax.while_loop(outer_cond, outer, (0, 0))

            pltpu.sync_copy(cu_v, cu_hbm)
            pltpu.sync_copy(cv_v, cv_hbm)
            pltpu.sync_copy(deg_v, deg_hbm)

    return pl.kernel(
        body,
        out_type=(
            jax.ShapeDtypeStruct((n_pad,), jnp.int32),
            jax.ShapeDtypeStruct((n_pad,), jnp.int32),
            jax.ShapeDtypeStruct((n_pad,), jnp.float32),
        ),
        mesh=mesh,
        compiler_params=pltpu.CompilerParams(needs_layout_passes=False),
        scratch_types=[
            pltpu.VMEM((n_pad,), jnp.int32),
            pltpu.VMEM((n_pad,), jnp.int32),
            pltpu.VMEM((n_pad,), jnp.int32),
            pltpu.VMEM((n_pad,), jnp.int32),
            pltpu.VMEM((n_pad,), jnp.float32),
            pltpu.VMEM((CH,), jnp.int32),
            pltpu.VMEM((CH,), jnp.int32),
            pltpu.SemaphoreType.DMA,
        ],
    )(u_s, v_s)


def _edge_agg(edge_attr, w_flat, u_idx, v_idx, zeros, n_pad):
    M, D = edge_attr.shape
    CH = 80
    NW = 32
    epw = M // NW
    assert epw % CH == 0
    nch = epw // CH
    rows_per_sub = n_pad // 16
    K8 = D // 16

    mesh = plsc.VectorSubcoreMesh(core_axis_name="c", subcore_axis_name="s")

    def body(ea_hbm, w_hbm, u_hbm, v_hbm, z_hbm, acc2_hbm,
             acc_sh, ea_v, w_v, iu_v, iv_v, sem):
        cid = lax.axis_index("c")
        sid = lax.axis_index("s")
        wid = sid * 2 + cid
        rows0 = sid * rows_per_sub

        pltpu.sync_copy(z_hbm.at[pl.ds(rows0, rows_per_sub)],
                        acc_sh.at[pl.ds(rows0, rows_per_sub)])
        plsc.subcore_barrier()

        base_w = wid * epw

        def chunk(ci, carry):
            base = pl.multiple_of(base_w + ci * CH, CH)
            pltpu.async_copy(ea_hbm.at[pl.ds(base, CH)], ea_v, sem).wait()
            pltpu.async_copy(w_hbm.at[pl.ds(base, CH)], w_v, sem).wait()
            pltpu.async_copy(u_hbm.at[pl.ds(base, CH)], iu_v, sem).wait()
            pltpu.async_copy(v_hbm.at[pl.ds(base, CH)], iv_v, sem).wait()

            def grp(g, carry2):
                w16 = w_v[pl.ds(g * 16, 16)]
                row0 = g * 16
                for r in range(16):
                    wv = jnp.full((16,), w16[r], jnp.float32)
                    row = row0 + r
                    for k in range(K8):
                        sl = pl.ds(k * 16, 16)
                        ea_v[row, sl] = ea_v[row, sl] * wv
                return carry2

            lax.fori_loop(0, CH // 16, grp, 0)
            pltpu.async_copy(ea_v, acc_sh.at[iu_v], sem, add=True).wait()
            pltpu.async_copy(ea_v, acc_sh.at[iv_v], sem, add=True).wait()
            return carry

        lax.fori_loop(0, nch, chunk, 0)
        plsc.subcore_barrier()

        off = cid * n_pad + rows0
        pltpu.sync_copy(acc_sh.at[pl.ds(rows0, rows_per_sub)],
                        acc2_hbm.at[pl.ds(off, rows_per_sub)])

    return pl.kernel(
        body,
        out_type=jax.ShapeDtypeStruct((2 * n_pad, D), jnp.float32),
        mesh=mesh,
        compiler_params=pltpu.CompilerParams(needs_layout_passes=False),
        scratch_types=[
            pltpu.VMEM_SHARED((n_pad, D), jnp.float32),
            pltpu.VMEM((CH, D), jnp.float32),
            pltpu.VMEM((CH,), jnp.float32),
            pltpu.VMEM((CH,), jnp.int32),
            pltpu.VMEM((CH,), jnp.int32),
            pltpu.SemaphoreType.DMA,
        ],
    )(edge_attr, w_flat, u_idx, v_idx, zeros)


def _mm_dis(x_pad, acc2, W, deg2d):
    NP, D = x_pad.shape
    RB = 1280
    grid = NP // RB
    half = NP // RB
    DRS = deg2d.shape

    def body(x_ref, a_ref, b_ref, w_ref, d_ref, xw_ref, dis_ref):
        x2 = x_ref[...] + a_ref[...] + b_ref[...]
        xw_ref[...] = jnp.dot(x2, w_ref[...],
                              preferred_element_type=jnp.float32)
        dis_ref[...] = lax.rsqrt(d_ref[...] + 1.0)

    return pl.pallas_call(
        body,
        grid=(grid,),
        in_specs=[
            pl.BlockSpec((RB, D), lambda i: (i, 0)),
            pl.BlockSpec((RB, D), lambda i: (i, 0)),
            pl.BlockSpec((RB, D), lambda i: (i + half, 0)),
            pl.BlockSpec((D, D), lambda i: (0, 0)),
            pl.BlockSpec(DRS, lambda i: (0, 0)),
        ],
        out_specs=[
            pl.BlockSpec((RB, D), lambda i: (i, 0)),
            pl.BlockSpec(DRS, lambda i: (0, 0)),
        ],
        out_shape=[
            jax.ShapeDtypeStruct((NP, D), jnp.float32),
            jax.ShapeDtypeStruct((NP // 128, 128), jnp.float32),
        ],
    )(x_pad, acc2, acc2, W, deg2d)


def _gcn_scatter(xw, dis, cu, cv, iota, zeros, n_pad):
    D = xw.shape[1]
    CH = 80
    NW = 32
    epw = n_pad // NW
    assert epw % CH == 0
    nch = epw // CH
    rows_per_sub = n_pad // 16
    K8 = D // 16

    mesh = plsc.VectorSubcoreMesh(core_axis_name="c", subcore_axis_name="s")

    def body(xw_hbm, dis_hbm, cu_hbm, cv_hbm, iota_hbm, z_hbm, acc2_hbm,
             acc_sh, dis_v, iu_v, iv_v, rows_v, sem):
        cid = lax.axis_index("c")
        sid = lax.axis_index("s")
        wid = sid * 2 + cid
        rows0 = sid * rows_per_sub

        pltpu.sync_copy(z_hbm.at[pl.ds(rows0, rows_per_sub)],
                        acc_sh.at[pl.ds(rows0, rows_per_sub)])
        pltpu.sync_copy(dis_hbm, dis_v)
        plsc.subcore_barrier()

        base_w = wid * epw

        def echunk(ci, carry):
            base = pl.multiple_of(base_w + ci * CH, CH)
            pltpu.async_copy(cu_hbm.at[pl.ds(base, CH)], iu_v, sem).wait()
            pltpu.async_copy(cv_hbm.at[pl.ds(base, CH)], iv_v, sem).wait()
            pltpu.async_copy(xw_hbm.at[iu_v], rows_v, sem).wait()

            def grp(g, carry2):
                iu16 = iu_v[pl.ds(g * 16, 16)]
                iv16 = iv_v[pl.ds(g * 16, 16)]
                sc16 = (plsc.load_gather(dis_v, [iu16])
                        * plsc.load_gather(dis_v, [iv16]))
                row0 = g * 16
                for r in range(16):
                    sv = jnp.full((16,), sc16[r], jnp.float32)
                    row = row0 + r
                    for k in range(K8):
                        sl = pl.ds(k * 16, 16)
                        rows_v[row, sl] = rows_v[row, sl] * sv
                return carry2

            lax.fori_loop(0, CH // 16, grp, 0)
            pltpu.async_copy(rows_v, acc_sh.at[iv_v], sem, add=True).wait()
            return carry

        lax.fori_loop(0, nch, echunk, 0)

        def schunk(ci, carry):
            base = pl.multiple_of(base_w + ci * CH, CH)
            pltpu.async_copy(iota_hbm.at[pl.ds(base, CH)], iv_v, sem).wait()
            pltpu.async_copy(xw_hbm.at[pl.ds(base, CH)], rows_v, sem).wait()

            def grp(g, carry2):
                d16 = dis_v[pl.ds(base + g * 16, 16)]
                sc16 = d16 * d16
                row0 = g * 16
                for r in range(16):
                    sv = jnp.full((16,), sc16[r], jnp.float32)
                    row = row0 + r
                    for k in range(K8):
                        sl = pl.ds(k * 16, 16)
                        rows_v[row, sl] = rows_v[row, sl] * sv
                return carry2

            lax.fori_loop(0, CH // 16, grp, 0)
            pltpu.async_copy(rows_v, acc_sh.at[iv_v], sem, add=True).wait()
            return carry

        lax.fori_loop(0, nch, schunk, 0)
        plsc.subcore_barrier()

        off = cid * n_pad + rows0
        pltpu.sync_copy(acc_sh.at[pl.ds(rows0, rows_per_sub)],
                        acc2_hbm.at[pl.ds(off, rows_per_sub)])

    return pl.kernel(
        body,
        out_type=jax.ShapeDtypeStruct((2 * n_pad, D), jnp.float32),
        mesh=mesh,
        compiler_params=pltpu.CompilerParams(needs_layout_passes=False),
        scratch_types=[
            pltpu.VMEM_SHARED((n_pad, D), jnp.float32),
            pltpu.VMEM((n_pad,), jnp.float32),
            pltpu.VMEM((CH,), jnp.int32),
            pltpu.VMEM((CH,), jnp.int32),
            pltpu.VMEM((CH, D), jnp.float32),
            pltpu.SemaphoreType.DMA,
        ],
    )(xw, dis, cu, cv, iota, zeros)


def _finalize(acc2, bvec):
    NP2, D = acc2.shape
    NP = NP2 // 2
    RB = 1280
    grid = NP // RB
    half = NP // RB

    def body(c_ref, d_ref, b_ref, o_ref):
        o = c_ref[...] + d_ref[...] + b_ref[...]
        o_ref[...] = jnp.where(o >= 0, o, o * 0.01)

    return pl.pallas_call(
        body,
        grid=(grid,),
        in_specs=[
            pl.BlockSpec((RB, D), lambda i: (i, 0)),
            pl.BlockSpec((RB, D), lambda i: (i + half, 0)),
            pl.BlockSpec((1, D), lambda i: (0, 0)),
        ],
        out_specs=pl.BlockSpec((RB, D), lambda i: (i, 0)),
        out_shape=jax.ShapeDtypeStruct((NP, D), jnp.float32),
    )(acc2, acc2, bvec.reshape(1, D))


def kernel(x, edge_index, edge_attr, Ws, bs, W, b):
    N, D = x.shape
    M = edge_index.shape[1]
    n_pad = ((N + 1 + 1279) // 1280) * 1280

    wpad = jnp.zeros((D, 8), jnp.float32).at[:, 0].set(Ws[0])
    s8 = _scores(edge_attr, wpad)
    s2d = s8[:, 0].reshape(M // 128, 128)
    w2d = _softmax(s2d, bs)
    w_flat = w2d.reshape(M)

    order = jnp.argsort(-w_flat)
    u_s = edge_index[0, order]
    v_s = edge_index[1, order]

    cu, cv, deg = _kruskal(u_s, v_s, N, n_pad)

    zeros = jnp.zeros((n_pad, D), jnp.float32)
    acc2 = _edge_agg(edge_attr, w_flat, edge_index[0], edge_index[1],
                     zeros, n_pad)

    x_pad = jnp.zeros((n_pad, D), jnp.float32).at[:N].set(x)
    xw, dis2d = _mm_dis(x_pad, acc2, W, deg.reshape(n_pad // 128, 128))

    iota = jnp.arange(n_pad, dtype=jnp.int32)
    gacc2 = _gcn_scatter(xw, dis2d.reshape(n_pad), cu, cv, iota,
                         zeros, n_pad)
    out_pad = _finalize(gacc2, b)
    return out_pad[:N]


# speedup vs baseline: 620.7887x; 1.6383x over previous
import functools

import jax
import jax.numpy as jnp
from jax import lax
from jax.experimental import pallas as pl
from jax.experimental.pallas import tpu as pltpu
from jax.experimental.pallas import tpu_sc as plsc


def _scores(edge_attr, wpad):
    M, D = edge_attr.shape
    RB = 2560
    grid = M // RB

    def body(ea_ref, w_ref, o_ref):
        o_ref[...] = jnp.dot(ea_ref[...], w_ref[...],
                             preferred_element_type=jnp.float32)

    return pl.pallas_call(
        body,
        grid=(grid,),
        in_specs=[
            pl.BlockSpec((RB, D), lambda i: (i, 0)),
            pl.BlockSpec((D, 8), lambda i: (0, 0)),
        ],
        out_specs=pl.BlockSpec((RB, 8), lambda i: (i, 0)),
        out_shape=jax.ShapeDtypeStruct((M, 8), jnp.float32),
    )(edge_attr, wpad)


def _softmax(s2d, bs):
    def body(s_ref, bs_ref, o_ref):
        s = s_ref[...] + bs_ref[0, 0]
        m = jnp.max(s)
        e = jnp.exp(s - m)
        o_ref[...] = e / jnp.sum(e)

    return pl.pallas_call(
        body,
        in_specs=[
            pl.BlockSpec(s2d.shape, lambda: (0, 0)),
            pl.BlockSpec(memory_space=pltpu.SMEM),
        ],
        out_specs=pl.BlockSpec(s2d.shape, lambda: (0, 0)),
        out_shape=jax.ShapeDtypeStruct(s2d.shape, jnp.float32),
    )(s2d, bs.reshape(1, 1))


def _kruskal(u_s, v_s, n_nodes, n_pad):
    M = u_s.shape[0]
    CH = 2000
    assert M % CH == 0
    n_stop = n_nodes - 1
    dummy = n_pad - 1

    mesh = plsc.VectorSubcoreMesh(core_axis_name="c", subcore_axis_name="s")

    def s0(vec):
        return vec[0]

    def rd(ref, i):
        return s0(plsc.load_gather(ref, [jnp.full((16,), i, jnp.int32)]))

    def body(u_hbm, v_hbm, cu_hbm, cv_hbm, deg_hbm,
             parent, rank, cu_v, cv_v, deg_v, uc, vc, sem):
        cid = lax.axis_index("c")
        sid = lax.axis_index("s")
        lane0 = lax.iota(jnp.int32, 16) == 0

        def wr(ref, i, val):
            plsc.store_scatter(ref, [jnp.full((16,), i, jnp.int32)],
                               jnp.full((16,), val, ref.dtype), mask=lane0)

        @pl.when((cid == 0) & (sid == 0))
        def _():
            iota16 = lax.iota(jnp.int32, 16)
            zero16f = jnp.zeros((16,), jnp.float32)
            zero16i = jnp.zeros((16,), jnp.int32)
            dummy16 = jnp.full((16,), dummy, jnp.int32)

            def init_body(i, carry):
                base = i * 16
                parent[pl.ds(base, 16)] = iota16 + base
                rank[pl.ds(base, 16)] = zero16i
                cu_v[pl.ds(base, 16)] = dummy16
                cv_v[pl.ds(base, 16)] = dummy16
                deg_v[pl.ds(base, 16)] = zero16f
                return carry

            lax.fori_loop(0, n_pad // 16, init_body, 0)

            def find(n):
                p = rd(parent, n)

                def cond(c):
                    node, par = c
                    return par != node

                def step(c):
                    node, par = c
                    gp = rd(parent, par)
                    wr(parent, node, gp)
                    return par, gp

                node, _ = lax.while_loop(cond, step, (n, p))
                return node

            def vfind(n16):
                def vcond(c2):
                    n, p = c2
                    return jnp.any(p != n)

                def vstep(c2):
                    n, p = c2
                    return p, plsc.load_gather(parent, [p])

                p0 = plsc.load_gather(parent, [n16])
                n, _ = lax.while_loop(vcond, vstep, (n16, p0))
                return n

            def edge_seq(base, cnt_in):
                def edge_body(jj, cnt2):
                    j = base + jj
                    u = rd(uc, j)
                    v = rd(vc, j)
                    ru = find(u)
                    rv = find(v)
                    acc = ru != rv

                    @pl.when(acc)
                    def _():
                        ra = rd(rank, ru)
                        rb = rd(rank, rv)
                        child = jnp.where(ra < rb, ru, rv)
                        root = jnp.where(ra < rb, rv, ru)
                        wr(parent, child, root)

                        @pl.when(ra == rb)
                        def _():
                            wr(rank, ru, ra + 1)

                        wr(cu_v, cnt2, u)
                        wr(cv_v, cnt2, v)
                        wr(deg_v, v, rd(deg_v, v) + 1.0)

                    return cnt2 + acc.astype(jnp.int32)

                return lax.fori_loop(0, 16, edge_body, cnt_in)

            def inner(c):
                g, cnt = c
                off = g * 16
                u16 = uc[pl.ds(off, 16)]
                v16 = vc[pl.ds(off, 16)]
                may_accept = jnp.any(vfind(u16) != vfind(v16))
                cnt = lax.cond(may_accept,
                               lambda ci: edge_seq(off, ci),
                               lambda ci: ci, cnt)
                return g + 1, cnt

            def inner_cond(c):
                g, cnt = c
                return (g < CH // 16) & (cnt < n_stop)

            def outer(c):
                pos, cnt = c
                pos = pl.multiple_of(pos, CH)
                pltpu.async_copy(u_hbm.at[pl.ds(pos, CH)], uc, sem).wait()
                pltpu.async_copy(v_hbm.at[pl.ds(pos, CH)], vc, sem).wait()
                _, cnt = lax.while_loop(inner_cond, inner, (0, cnt))
                return pos + CH, cnt

            def outer_cond(c):
                pos, cnt = c
                return (pos < M) & (cnt < n_stop)

            lax.while_loop(outer_cond, outer, (0, 0))

            pltpu.sync_copy(cu_v, cu_hbm)
            pltpu.sync_copy(cv_v, cv_hbm)
            pltpu.sync_copy(deg_v, deg_hbm)

    return pl.kernel(
        body,
        out_type=(
            jax.ShapeDtypeStruct((n_pad,), jnp.int32),
            jax.ShapeDtypeStruct((n_pad,), jnp.int32),
            jax.ShapeDtypeStruct((n_pad,), jnp.float32),
        ),
        mesh=mesh,
        compiler_params=pltpu.CompilerParams(needs_layout_passes=False),
        scratch_types=[
            pltpu.VMEM((n_pad,), jnp.int32),
            pltpu.VMEM((n_pad,), jnp.int32),
            pltpu.VMEM((n_pad,), jnp.int32),
            pltpu.VMEM((n_pad,), jnp.int32),
            pltpu.VMEM((n_pad,), jnp.float32),
            pltpu.VMEM((CH,), jnp.int32),
            pltpu.VMEM((CH,), jnp.int32),
            pltpu.SemaphoreType.DMA,
        ],
    )(u_s, v_s)


def _edge_agg(edge_attr, w_flat, u_idx, v_idx, zeros, n_pad):
    M, D = edge_attr.shape
    CH = 80
    NW = 32
    epw = M // NW
    assert epw % CH == 0
    nch = epw // CH
    rows_per_sub = n_pad // 16
    K8 = D // 16

    mesh = plsc.VectorSubcoreMesh(core_axis_name="c", subcore_axis_name="s")

    def body(ea_hbm, w_hbm, u_hbm, v_hbm, z_hbm, acc2_hbm,
             acc_sh, ea_v, w_v, iu_v, iv_v, sem):
        cid = lax.axis_index("c")
        sid = lax.axis_index("s")
        wid = sid * 2 + cid
        rows0 = sid * rows_per_sub

        pltpu.sync_copy(z_hbm.at[pl.ds(rows0, rows_per_sub)],
                        acc_sh.at[pl.ds(rows0, rows_per_sub)])
        plsc.subcore_barrier()

        base_w = wid * epw

        def chunk(ci, carry):
            base = pl.multiple_of(base_w + ci * CH, CH)
            pltpu.async_copy(ea_hbm.at[pl.ds(base, CH)], ea_v, sem).wait()
            pltpu.async_copy(w_hbm.at[pl.ds(base, CH)], w_v, sem).wait()
            pltpu.async_copy(u_hbm.at[pl.ds(base, CH)], iu_v, sem).wait()
            pltpu.async_copy(v_hbm.at[pl.ds(base, CH)], iv_v, sem).wait()

            def grp(g, carry2):
                w16 = w_v[pl.ds(g * 16, 16)]
                row0 = g * 16
                for r in range(16):
                    wv = jnp.full((16,), w16[r], jnp.float32)
                    row = row0 + r
                    for k in range(K8):
                        sl = pl.ds(k * 16, 16)
                        ea_v[row, sl] = ea_v[row, sl] * wv
                return carry2

            lax.fori_loop(0, CH // 16, grp, 0)
            pltpu.async_copy(ea_v, acc_sh.at[iu_v], sem, add=True).wait()
            pltpu.async_copy(ea_v, acc_sh.at[iv_v], sem, add=True).wait()
            return carry

        lax.fori_loop(0, nch, chunk, 0)
        plsc.subcore_barrier()

        off = cid * n_pad + rows0
        pltpu.sync_copy(acc_sh.at[pl.ds(rows0, rows_per_sub)],
                        acc2_hbm.at[pl.ds(off, rows_per_sub)])

    return pl.kernel(
        body,
        out_type=jax.ShapeDtypeStruct((2 * n_pad, D), jnp.float32),
        mesh=mesh,
        compiler_params=pltpu.CompilerParams(needs_layout_passes=False),
        scratch_types=[
            pltpu.VMEM_SHARED((n_pad, D), jnp.float32),
            pltpu.VMEM((CH, D), jnp.float32),
            pltpu.VMEM((CH,), jnp.float32),
            pltpu.VMEM((CH,), jnp.int32),
            pltpu.VMEM((CH,), jnp.int32),
            pltpu.SemaphoreType.DMA,
        ],
    )(edge_attr, w_flat, u_idx, v_idx, zeros)


def _mm_dis(x_pad, acc2, W, deg2d):
    NP, D = x_pad.shape
    RB = 1280
    grid = NP // RB
    half = NP // RB
    DRS = deg2d.shape

    def body(x_ref, a_ref, b_ref, w_ref, d_ref, xw_ref, dis_ref):
        x2 = x_ref[...] + a_ref[...] + b_ref[...]
        xw_ref[...] = jnp.dot(x2, w_ref[...],
                              preferred_element_type=jnp.float32)
        dis_ref[...] = lax.rsqrt(d_ref[...] + 1.0)

    return pl.pallas_call(
        body,
        grid=(grid,),
        in_specs=[
            pl.BlockSpec((RB, D), lambda i: (i, 0)),
            pl.BlockSpec((RB, D), lambda i: (i, 0)),
            pl.BlockSpec((RB, D), lambda i: (i + half, 0)),
            pl.BlockSpec((D, D), lambda i: (0, 0)),
            pl.BlockSpec(DRS, lambda i: (0, 0)),
        ],
        out_specs=[
            pl.BlockSpec((RB, D), lambda i: (i, 0)),
            pl.BlockSpec(DRS, lambda i: (0, 0)),
        ],
        out_shape=[
            jax.ShapeDtypeStruct((NP, D), jnp.float32),
            jax.ShapeDtypeStruct((NP // 128, 128), jnp.float32),
        ],
    )(x_pad, acc2, acc2, W, deg2d)


def _gcn_scatter(xw, dis, cu, cv, iota, zeros, n_pad):
    D = xw.shape[1]
    CH = 80
    NW = 32
    epw = n_pad // NW
    assert epw % CH == 0
    nch = epw // CH
    rows_per_sub = n_pad // 16
    K8 = D // 16

    mesh = plsc.VectorSubcoreMesh(core_axis_name="c", subcore_axis_name="s")

    def body(xw_hbm, dis_hbm, cu_hbm, cv_hbm, iota_hbm, z_hbm, acc2_hbm,
             acc_sh, dis_v, iu_v, iv_v, rows_v, sem):
        cid = lax.axis_index("c")
        sid = lax.axis_index("s")
        wid = sid * 2 + cid
        rows0 = sid * rows_per_sub

        pltpu.sync_copy(z_hbm.at[pl.ds(rows0, rows_per_sub)],
                        acc_sh.at[pl.ds(rows0, rows_per_sub)])
        pltpu.sync_copy(dis_hbm, dis_v)
        plsc.subcore_barrier()

        base_w = wid * epw

        def echunk(ci, carry):
            base = pl.multiple_of(base_w + ci * CH, CH)
            pltpu.async_copy(cu_hbm.at[pl.ds(base, CH)], iu_v, sem).wait()
            pltpu.async_copy(cv_hbm.at[pl.ds(base, CH)], iv_v, sem).wait()
            pltpu.async_copy(xw_hbm.at[iu_v], rows_v, sem).wait()

            def grp(g, carry2):
                iu16 = iu_v[pl.ds(g * 16, 16)]
                iv16 = iv_v[pl.ds(g * 16, 16)]
                sc16 = (plsc.load_gather(dis_v, [iu16])
                        * plsc.load_gather(dis_v, [iv16]))
                row0 = g * 16
                for r in range(16):
                    sv = jnp.full((16,), sc16[r], jnp.float32)
                    row = row0 + r
                    for k in range(K8):
                        sl = pl.ds(k * 16, 16)
                        rows_v[row, sl] = rows_v[row, sl] * sv
                return carry2

            lax.fori_loop(0, CH // 16, grp, 0)
            pltpu.async_copy(rows_v, acc_sh.at[iv_v], sem, add=True).wait()
            return carry

        lax.fori_loop(0, nch, echunk, 0)

        def schunk(ci, carry):
            base = pl.multiple_of(base_w + ci * CH, CH)
            pltpu.async_copy(iota_hbm.at[pl.ds(base, CH)], iv_v, sem).wait()
            pltpu.async_copy(xw_hbm.at[pl.ds(base, CH)], rows_v, sem).wait()

            def grp(g, carry2):
                d16 = dis_v[pl.ds(base + g * 16, 16)]
                sc16 = d16 * d16
                row0 = g * 16
                for r in range(16):
                    sv = jnp.full((16,), sc16[r], jnp.float32)
                    row = row0 + r
                    for k in range(K8):
                        sl = pl.ds(k * 16, 16)
                        rows_v[row, sl] = rows_v[row, sl] * sv
                return carry2

            lax.fori_loop(0, CH // 16, grp, 0)
            pltpu.async_copy(rows_v, acc_sh.at[iv_v], sem, add=True).wait()
            return carry

        lax.fori_loop(0, nch, schunk, 0)
        plsc.subcore_barrier()

        off = cid * n_pad + rows0
        pltpu.sync_copy(acc_sh.at[pl.ds(rows0, rows_per_sub)],
                        acc2_hbm.at[pl.ds(off, rows_per_sub)])

    return pl.kernel(
        body,
        out_type=jax.ShapeDtypeStruct((2 * n_pad, D), jnp.float32),
        mesh=mesh,
        compiler_params=pltpu.CompilerParams(needs_layout_passes=False),
        scratch_types=[
            pltpu.VMEM_SHARED((n_pad, D), jnp.float32),
            pltpu.VMEM((n_pad,), jnp.float32),
            pltpu.VMEM((CH,), jnp.int32),
            pltpu.VMEM((CH,), jnp.int32),
            pltpu.VMEM((CH, D), jnp.float32),
            pltpu.SemaphoreType.DMA,
        ],
    )(xw, dis, cu, cv, iota, zeros)


def _finalize(acc2, bvec):
    NP2, D = acc2.shape
    NP = NP2 // 2
    RB = 1280
    grid = NP // RB
    half = NP // RB

    def body(c_ref, d_ref, b_ref, o_ref):
        o = c_ref[...] + d_ref[...] + b_ref[...]
        o_ref[...] = jnp.where(o >= 0, o, o * 0.01)

    return pl.pallas_call(
        body,
        grid=(grid,),
        in_specs=[
            pl.BlockSpec((RB, D), lambda i: (i, 0)),
            pl.BlockSpec((RB, D), lambda i: (i + half, 0)),
            pl.BlockSpec((1, D), lambda i: (0, 0)),
        ],
        out_specs=pl.BlockSpec((RB, D), lambda i: (i, 0)),
        out_shape=jax.ShapeDtypeStruct((NP, D), jnp.float32),
    )(acc2, acc2, bvec.reshape(1, D))


def kernel(x, edge_index, edge_attr, Ws, bs, W, b):
    N, D = x.shape
    M = edge_index.shape[1]
    n_pad = ((N + 1 + 1279) // 1280) * 1280

    wpad = jnp.zeros((D, 8), jnp.float32).at[:, 0].set(Ws[0])
    s8 = _scores(edge_attr, wpad)
    s2d = s8[:, 0].reshape(M // 128, 128)
    w2d = _softmax(s2d, bs)
    w_flat = w2d.reshape(M)

    order = jnp.argsort(-w_flat)
    u_s = edge_index[0, order]
    v_s = edge_index[1, order]

    cu, cv, deg = _kruskal(u_s, v_s, N, n_pad)

    zeros = jnp.zeros((n_pad, D), jnp.float32)
    acc2 = _edge_agg(edge_attr, w_flat, edge_index[0], edge_index[1],
                     zeros, n_pad)

    x_pad = jnp.zeros((n_pad, D), jnp.float32).at[:N].set(x)
    xw, dis2d = _mm_dis(x_pad, acc2, W, deg.reshape(n_pad // 128, 128))

    iota = jnp.arange(n_pad, dtype=jnp.int32)
    gacc2 = _gcn_scatter(xw, dis2d.reshape(n_pad), cu, cv, iota,
                         zeros, n_pad)
    out_pad = _finalize(gacc2, b)
    return out_pad[:N]
